# baseline (speedup 1.0000x reference)
import jax
import jax.numpy as jnp
from jax import lax
from jax.experimental import pallas as pl
from jax.experimental.pallas import tpu as pltpu
from jax.experimental.pallas import tpu_sc as plsc

N_NODES = 10000
N_EDGES = 320000
D_NODE = 128
D_EDGE = 16
D_GLOB = 32
N_GRAPHS = 4

N_WORKERS = 32
EDGES_PER_TILE = N_EDGES // N_WORKERS
CHUNK = 80
N_CHUNKS = EDGES_PER_TILE // CHUNK
ACC_ROWS = 10240
TILE_ROWS = ACC_ROWS // 16

NODE_BLK = 1000

_F32 = jnp.float32
_I32 = jnp.int32


def _sc_edge_body(a2_hbm, b_hbm, src_hbm, dst_hbm, bnd_hbm,
                  e_hbm, msg_hbm, cnt_hbm, egp_hbm, ecp_hbm,
                  src_v, dst_v, rowt, e_buf, b_v, ones_v, bnd_v,
                  egt, ect, msg_sh, cnt_sh, sem_g, sem_e, sem_m, sem_c):
    cid = lax.axis_index("c")
    sid = lax.axis_index("s")
    wid = cid * 16 + sid

    zero16 = jnp.zeros((16,), _F32)
    iota16 = lax.iota(_I32, 16)

    def _z_ones(i, c):
        ones_v[i, :] = zero16
        return c
    lax.fori_loop(0, CHUNK, _z_ones, 0)

    def _z_ebuf(i, c):
        for j in range(8):
            e_buf[i, pl.ds(j * 16, 16)] = zero16
        return c
    lax.fori_loop(0, CHUNK, _z_ebuf, 0)

    for g in range(N_GRAPHS):
        for j in range(8):
            egt[g, pl.ds(j * 16, 16)] = zero16
        ect[g, :] = zero16

    pltpu.sync_copy(bnd_hbm, bnd_v)

    rbase = sid * TILE_ROWS
    obase = cid * ACC_ROWS + rbase
    for k in range(8):
        pltpu.sync_copy(e_buf.at[pl.ds(0, 80)],
                        msg_sh.at[pl.ds(rbase + k * 80, 80)])
        pltpu.sync_copy(ones_v.at[pl.ds(0, 80)],
                        cnt_sh.at[pl.ds(rbase + k * 80, 80)])

    lane1 = jnp.where(iota16 == 0, jnp.float32(1.0), jnp.float32(0.0))

    def _s_ones(i, c):
        ones_v[i, :] = lane1
        return c
    lax.fori_loop(0, CHUNK, _s_ones, 0)

    plsc.subcore_barrier()

    ebase = wid * EDGES_PER_TILE
    one16 = jnp.full((16,), 1.0, _F32)

    def _chunk(t, c):
        base = ebase + t * CHUNK
        pltpu.sync_copy(src_hbm.at[pl.ds(base, CHUNK)], src_v)
        pltpu.sync_copy(dst_hbm.at[pl.ds(base, CHUNK)], dst_v.at[0])
        gcp = pltpu.async_copy(a2_hbm.at[src_v], e_buf, sem_g)
        pltpu.sync_copy(b_hbm.at[pl.ds(base, CHUNK)], b_v)

        c1v = bnd_v[0, pl.ds(0, 16)]
        c2v = bnd_v[1, pl.ds(0, 16)]
        c3v = bnd_v[2, pl.ds(0, 16)]
        i1 = jnp.full((16,), 1, _I32)
        i0 = jnp.full((16,), 0, _I32)
        for grp in range(CHUNK // 16):
            s16 = src_v[pl.ds(grp * 16, 16)]
            g16 = (jnp.where(s16 >= c1v, i1, i0)
                   + jnp.where(s16 >= c2v, i1, i0)
                   + jnp.where(s16 >= c3v, i1, i0))
            for l in range(16):
                plsc.store_scatter(rowt, [iota16 * 16 + (grp * 256 + l)], g16)
            plsc.addupdate_scatter(ect, [g16, iota16], one16)

        gcp.wait()

        def _rows(i, cc):
            row16 = rowt[pl.ds(i * 16, 16)]
            for j in range(8):
                v = e_buf[i, pl.ds(j * 16, 16)] + b_v[i, pl.ds(j * 16, 16)]
                v = jnp.maximum(v, 0.0)
                e_buf[i, pl.ds(j * 16, 16)] = v
                plsc.addupdate_scatter(egt, [row16, iota16 + j * 16], v)
            return cc
        lax.fori_loop(0, CHUNK, _rows, 0)

        cp_e = pltpu.async_copy(e_buf, e_hbm.at[pl.ds(base, CHUNK)], sem_e)
        cp_m = pltpu.async_copy(e_buf, msg_sh.at[dst_v.at[0]], sem_m, add=True)
        cp_c = pltpu.async_copy(ones_v, cnt_sh.at[dst_v.at[0]], sem_c, add=True)
        cp_e.wait()
        cp_m.wait()
        cp_c.wait()
        return c

    lax.fori_loop(0, N_CHUNKS, _chunk, 0)

    plsc.subcore_barrier()

    pltpu.sync_copy(egt, egp_hbm.at[wid])
    pltpu.sync_copy(ect, ecp_hbm.at[wid])
    for k in range(8):
        pltpu.sync_copy(msg_sh.at[pl.ds(rbase + k * 80, 80)], e_buf)
        pltpu.sync_copy(e_buf, msg_hbm.at[pl.ds(obase + k * 80, 80)])
        pltpu.sync_copy(cnt_sh.at[pl.ds(rbase + k * 80, 80)], ones_v)
        pltpu.sync_copy(ones_v, cnt_hbm.at[pl.ds(obase + k * 80, 80)])


def _sc_edge(a2, b, src, dst, bnd):
    f = pl.kernel(
        _sc_edge_body,
        out_type=[
            jax.ShapeDtypeStruct((N_EDGES, D_NODE), _F32),
            jax.ShapeDtypeStruct((2 * ACC_ROWS, D_NODE), _F32),
            jax.ShapeDtypeStruct((2 * ACC_ROWS, 16), _F32),
            jax.ShapeDtypeStruct((N_WORKERS, N_GRAPHS, D_NODE), _F32),
            jax.ShapeDtypeStruct((N_WORKERS, N_GRAPHS, 16), _F32),
        ],
        mesh=plsc.VectorSubcoreMesh(core_axis_name="c", subcore_axis_name="s"),
        compiler_params=pltpu.CompilerParams(use_tc_tiling_on_sc=False,
                                            needs_layout_passes=False),
        scratch_types=[
            pltpu.VMEM((CHUNK,), _I32),
            pltpu.VMEM((1, CHUNK), _I32),
            pltpu.VMEM((CHUNK * 16,), _I32),
            pltpu.VMEM((CHUNK, D_NODE), _F32),
            pltpu.VMEM((CHUNK, D_NODE), _F32),
            pltpu.VMEM((CHUNK, 16), _F32),
            pltpu.VMEM((8, 128), _I32),
            pltpu.VMEM((N_GRAPHS, D_NODE), _F32),
            pltpu.VMEM((N_GRAPHS, 16), _F32),
            pltpu.VMEM_SHARED((ACC_ROWS, D_NODE), _F32),
            pltpu.VMEM_SHARED((ACC_ROWS, 16), _F32),
            pltpu.SemaphoreType.DMA,
            pltpu.SemaphoreType.DMA,
            pltpu.SemaphoreType.DMA,
            pltpu.SemaphoreType.DMA,
        ],
    )
    return f(a2, b, src, dst, bnd)


def _a2_body(x_ref, bcol_ref, bfull_ref, u_ref, we1_ref, we3_ref, be_ref,
             out_ref, bnd_ref):
    i = pl.program_id(0)
    oh = (lax.broadcasted_iota(_I32, (NODE_BLK, N_GRAPHS), 1)
          == bcol_ref[...]).astype(_F32)
    uwe3 = jnp.dot(u_ref[...], we3_ref[...], preferred_element_type=_F32)
    out_ref[...] = (jnp.dot(x_ref[...], we1_ref[...], preferred_element_type=_F32)
                    + jnp.dot(oh, uwe3, preferred_element_type=_F32)
                    + be_ref[...])

    @pl.when(i == 0)
    def _():
        bf = bfull_ref[...].astype(_F32)
        c1 = jnp.sum((bf < 1.0).astype(_F32))
        c2 = jnp.sum((bf < 2.0).astype(_F32))
        c3 = jnp.sum((bf < 3.0).astype(_F32))
        row = lax.broadcasted_iota(_I32, (8, 128), 0)
        c1i = c1.astype(_I32)
        c2i = c2.astype(_I32)
        c3i = c3.astype(_I32)
        bnd_ref[...] = jnp.where(
            row == 0, c1i, jnp.where(row == 1, c2i,
                                     jnp.where(row == 2, c3i, 0)))


def _b_body(ea_ref, we2_ref, out_ref):
    out_ref[...] = jnp.dot(ea_ref[...], we2_ref[...], preferred_element_type=_F32)


def _node_body(x_ref, m0_ref, m1_ref, c0_ref, c1_ref, bcol_ref,
               u_ref, wn1_ref, wn2_ref, wn3_ref, bn_ref,
               xout_ref, xg_ref, nc_ref):
    i = pl.program_id(0)
    cnt = c0_ref[:, 0:1] + c1_ref[:, 0:1]
    mean = (m0_ref[...] + m1_ref[...]) / jnp.maximum(cnt, 1.0)
    oh = (lax.broadcasted_iota(_I32, (NODE_BLK, N_GRAPHS), 1)
          == bcol_ref[...]).astype(_F32)
    uwn3 = jnp.dot(u_ref[...], wn3_ref[...], preferred_element_type=_F32)
    acc = (jnp.dot(x_ref[...], wn1_ref[...], preferred_element_type=_F32)
           + jnp.dot(mean, wn2_ref[...], preferred_element_type=_F32)
           + jnp.dot(oh, uwn3, preferred_element_type=_F32)
           + bn_ref[...])
    xo = jnp.maximum(acc, 0.0)
    xout_ref[...] = xo

    @pl.when(i == 0)
    def _():
        xg_ref[...] = jnp.zeros_like(xg_ref)
        nc_ref[...] = jnp.zeros_like(nc_ref)

    dn = (((0,), (0,)), ((), ()))
    xg_ref[...] += lax.dot_general(oh, xo, dn, preferred_element_type=_F32)
    nc_ref[...] += lax.dot_general(oh, jnp.ones_like(xo), dn,
                                   preferred_element_type=_F32)


def _glob_body(egp_ref, ecp_ref, xg_ref, nc_ref,
               u_ref, wg1_ref, wg2_ref, wg3_ref, bg_ref, uout_ref):
    eg = jnp.sum(egp_ref[...], axis=0)
    ec = jnp.sum(jnp.sum(ecp_ref[...], axis=0), axis=1, keepdims=True)
    egm = eg / jnp.maximum(ec, 1.0)
    xgm = xg_ref[...] / jnp.maximum(nc_ref[...], 1.0)
    uo = (jnp.dot(egm, wg1_ref[...], preferred_element_type=_F32)
          + jnp.dot(xgm, wg2_ref[...], preferred_element_type=_F32)
          + jnp.dot(u_ref[...], wg3_ref[...], preferred_element_type=_F32)
          + bg_ref[...])
    uout_ref[...] = jnp.maximum(uo, 0.0)


def kernel(x, edge_attr, edge_index, u, batch, W_e, b_e, W_n, b_n, W_g, b_g):
    src = edge_index[0].astype(_I32)
    dst = edge_index[1].astype(_I32)
    bcol = batch.astype(_I32).reshape(N_NODES, 1)

    we1 = W_e[:D_NODE]
    we2 = W_e[D_NODE:D_NODE + D_EDGE]
    we3 = W_e[D_NODE + D_EDGE:]
    wn1 = W_n[:D_NODE]
    wn2 = W_n[D_NODE:2 * D_NODE]
    wn3 = W_n[2 * D_NODE:]
    wg1 = W_g[:D_NODE]
    wg2 = W_g[D_NODE:2 * D_NODE]
    wg3 = W_g[2 * D_NODE:]
    be = b_e.reshape(1, D_NODE)
    bn = b_n.reshape(1, D_NODE)
    bg = b_g.reshape(1, D_NODE)

    n_grid = N_NODES // NODE_BLK
    e_grid = N_EDGES // 4000

    full = lambda s: pl.BlockSpec(s, lambda i: tuple(0 for _ in s))

    a2, bnd = pl.pallas_call(
        _a2_body,
        grid=(n_grid,),
        in_specs=[
            pl.BlockSpec((NODE_BLK, D_NODE), lambda i: (i, 0)),
            pl.BlockSpec((NODE_BLK, 1), lambda i: (i, 0)),
            full((N_NODES, 1)),
            full((N_GRAPHS, D_GLOB)),
            full((D_NODE, D_NODE)),
            full((D_GLOB, D_NODE)),
            full((1, D_NODE)),
        ],
        out_specs=[
            pl.BlockSpec((NODE_BLK, D_NODE), lambda i: (i, 0)),
            pl.BlockSpec((8, 128), lambda i: (0, 0)),
        ],
        out_shape=[
            jax.ShapeDtypeStruct((N_NODES, D_NODE), _F32),
            jax.ShapeDtypeStruct((8, 128), _I32),
        ],
    )(x, bcol, bcol, u, we1, we3, be)

    b_edges = pl.pallas_call(
        _b_body,
        grid=(e_grid,),
        in_specs=[
            pl.BlockSpec((4000, D_EDGE), lambda i: (i, 0)),
            full((D_EDGE, D_NODE)),
        ],
        out_specs=pl.BlockSpec((4000, D_NODE), lambda i: (i, 0)),
        out_shape=jax.ShapeDtypeStruct((N_EDGES, D_NODE), _F32),
    )(edge_attr, we2)

    e, msg_p, cnt_p, egp, ecp = _sc_edge(a2, b_edges, src, dst, bnd)
    msg_p = msg_p.reshape(2, ACC_ROWS, D_NODE)[:, :N_NODES]
    cnt_p = cnt_p.reshape(2, ACC_ROWS, 16)[:, :N_NODES]

    x_out, xg, nc = pl.pallas_call(
        _node_body,
        grid=(n_grid,),
        in_specs=[
            pl.BlockSpec((NODE_BLK, D_NODE), lambda i: (i, 0)),
            pl.BlockSpec((NODE_BLK, D_NODE), lambda i: (i, 0)),
            pl.BlockSpec((NODE_BLK, D_NODE), lambda i: (i, 0)),
            pl.BlockSpec((NODE_BLK, 16), lambda i: (i, 0)),
            pl.BlockSpec((NODE_BLK, 16), lambda i: (i, 0)),
            pl.BlockSpec((NODE_BLK, 1), lambda i: (i, 0)),
            full((N_GRAPHS, D_GLOB)),
            full((D_NODE, D_NODE)),
            full((D_NODE, D_NODE)),
            full((D_GLOB, D_NODE)),
            full((1, D_NODE)),
        ],
        out_specs=[
            pl.BlockSpec((NODE_BLK, D_NODE), lambda i: (i, 0)),
            pl.BlockSpec((N_GRAPHS, D_NODE), lambda i: (0, 0)),
            pl.BlockSpec((N_GRAPHS, D_NODE), lambda i: (0, 0)),
        ],
        out_shape=[
            jax.ShapeDtypeStruct((N_NODES, D_NODE), _F32),
            jax.ShapeDtypeStruct((N_GRAPHS, D_NODE), _F32),
            jax.ShapeDtypeStruct((N_GRAPHS, D_NODE), _F32),
        ],
    )(x, msg_p[0], msg_p[1], cnt_p[0], cnt_p[1], bcol, u, wn1, wn2, wn3, bn)

    u_out = pl.pallas_call(
        _glob_body,
        grid=(1,),
        in_specs=[
            full((N_WORKERS, N_GRAPHS, D_NODE)),
            full((N_WORKERS, N_GRAPHS, 16)),
            full((N_GRAPHS, D_NODE)),
            full((N_GRAPHS, D_NODE)),
            full((N_GRAPHS, D_GLOB)),
            full((D_NODE, D_NODE)),
            full((D_NODE, D_NODE)),
            full((D_GLOB, D_NODE)),
            full((1, D_NODE)),
        ],
        out_specs=pl.BlockSpec((N_GRAPHS, D_NODE), lambda i: (0, 0)),
        out_shape=jax.ShapeDtypeStruct((N_GRAPHS, D_NODE), _F32),
    )(egp, ecp, xg, nc, u, wg1, wg2, wg3, bg)

    return (x_out, e, edge_index, u_out, batch)


# speedup vs baseline: 1.4346x; 1.4346x over previous
import jax
import jax.numpy as jnp
from jax import lax
from jax.experimental import pallas as pl
from jax.experimental.pallas import tpu as pltpu
from jax.experimental.pallas import tpu_sc as plsc

N_NODES = 10000
N_EDGES = 320000
D_NODE = 128
D_EDGE = 16
D_GLOB = 32
N_GRAPHS = 4

N_WORKERS = 32
EDGES_PER_TILE = N_EDGES // N_WORKERS
CHUNK = 80
N_CHUNKS = EDGES_PER_TILE // CHUNK
ACC_ROWS = 10240
TILE_ROWS = ACC_ROWS // 16

NODE_BLK = 1000

_F32 = jnp.float32
_I32 = jnp.int32


def _sc_edge_body(a2_hbm, b_hbm, src_hbm, dst_hbm, bnd_hbm,
                  e_hbm, msg_hbm, cnt_hbm,
                  src_v, dst_v, gidx, e_buf, b_v, ones_v, bnd_v,
                  msg_sh, cnt_sh, sem_g, sem_e, sem_m, sem_c, sem_x, sem_y):
    cid = lax.axis_index("c")
    sid = lax.axis_index("s")
    wid = cid * 16 + sid

    zero16 = jnp.zeros((16,), _F32)
    iota16 = lax.iota(_I32, 16)

    def _z_ones(i, c):
        ones_v[i, :] = zero16
        return c
    lax.fori_loop(0, CHUNK, _z_ones, 0)

    def _z_ebuf(i, c):
        for j in range(8):
            e_buf[i, pl.ds(j * 16, 16)] = zero16
        return c
    lax.fori_loop(0, CHUNK, _z_ebuf, 0)

    pltpu.sync_copy(bnd_hbm, bnd_v)

    rbase = sid * TILE_ROWS
    obase = cid * ACC_ROWS + rbase
    for k in range(8):
        pltpu.sync_copy(e_buf.at[pl.ds(0, 80)],
                        msg_sh.at[pl.ds(rbase + k * 80, 80)])
        pltpu.sync_copy(ones_v.at[pl.ds(0, 80)],
                        cnt_sh.at[pl.ds(rbase + k * 80, 80)])

    lane1 = jnp.where(iota16 == 0, jnp.float32(1.0), jnp.float32(0.0))

    def _s_ones(i, c):
        ones_v[i, :] = lane1
        return c
    lax.fori_loop(0, CHUNK, _s_ones, 0)

    plsc.subcore_barrier()

    ebase = wid * EDGES_PER_TILE
    one16 = jnp.full((16,), 1.0, _F32)

    def _chunk(t, c):
        base = ebase + t * CHUNK
        pltpu.sync_copy(src_hbm.at[pl.ds(base, CHUNK)], src_v)
        pltpu.sync_copy(dst_hbm.at[pl.ds(base, CHUNK)], dst_v.at[0])
        gcp = pltpu.async_copy(a2_hbm.at[src_v], e_buf, sem_g)
        pltpu.sync_copy(b_hbm.at[pl.ds(base, CHUNK)], b_v)

        c1v = bnd_v[0, pl.ds(0, 16)]
        c2v = bnd_v[1, pl.ds(0, 16)]
        c3v = bnd_v[2, pl.ds(0, 16)]
        i1 = jnp.full((16,), 1, _I32)
        i0 = jnp.full((16,), 0, _I32)
        gb = N_NODES + sid * 4
        for grp in range(CHUNK // 16):
            s16 = src_v[pl.ds(grp * 16, 16)]
            g16 = (jnp.where(s16 >= c1v, i1, i0)
                   + jnp.where(s16 >= c2v, i1, i0)
                   + jnp.where(s16 >= c3v, i1, i0))
            gidx[0, pl.ds(grp * 16, 16)] = g16 + gb

        gcp.wait()

        def _rows(i, cc):
            for j in range(8):
                v = e_buf[i, pl.ds(j * 16, 16)] + b_v[i, pl.ds(j * 16, 16)]
                v = jnp.maximum(v, 0.0)
                e_buf[i, pl.ds(j * 16, 16)] = v
            return cc
        lax.fori_loop(0, CHUNK, _rows, 0)

        cp_e = pltpu.async_copy(e_buf, e_hbm.at[pl.ds(base, CHUNK)], sem_e)
        cp_m = pltpu.async_copy(e_buf, msg_sh.at[dst_v.at[0]], sem_m, add=True)
        cp_c = pltpu.async_copy(ones_v, cnt_sh.at[dst_v.at[0]], sem_c, add=True)
        cp_x = pltpu.async_copy(e_buf, msg_sh.at[gidx.at[0]], sem_x, add=True)
        cp_y = pltpu.async_copy(ones_v, cnt_sh.at[gidx.at[0]], sem_y, add=True)
        cp_e.wait()
        cp_m.wait()
        cp_c.wait()
        cp_x.wait()
        cp_y.wait()
        return c

    lax.fori_loop(0, N_CHUNKS, _chunk, 0)

    plsc.subcore_barrier()

    for k in range(8):
        pltpu.sync_copy(msg_sh.at[pl.ds(rbase + k * 80, 80)], e_buf)
        pltpu.sync_copy(e_buf, msg_hbm.at[pl.ds(obase + k * 80, 80)])
        pltpu.sync_copy(cnt_sh.at[pl.ds(rbase + k * 80, 80)], ones_v)
        pltpu.sync_copy(ones_v, cnt_hbm.at[pl.ds(obase + k * 80, 80)])


def _sc_edge(a2, b, src, dst, bnd):
    f = pl.kernel(
        _sc_edge_body,
        out_type=[
            jax.ShapeDtypeStruct((N_EDGES, D_NODE), _F32),
            jax.ShapeDtypeStruct((2 * ACC_ROWS, D_NODE), _F32),
            jax.ShapeDtypeStruct((2 * ACC_ROWS, 16), _F32),
        ],
        mesh=plsc.VectorSubcoreMesh(core_axis_name="c", subcore_axis_name="s"),
        compiler_params=pltpu.CompilerParams(use_tc_tiling_on_sc=False,
                                            needs_layout_passes=False),
        scratch_types=[
            pltpu.VMEM((CHUNK,), _I32),
            pltpu.VMEM((1, CHUNK), _I32),
            pltpu.VMEM((1, CHUNK), _I32),
            pltpu.VMEM((CHUNK, D_NODE), _F32),
            pltpu.VMEM((CHUNK, D_NODE), _F32),
            pltpu.VMEM((CHUNK, 16), _F32),
            pltpu.VMEM((8, 128), _I32),
            pltpu.VMEM_SHARED((ACC_ROWS, D_NODE), _F32),
            pltpu.VMEM_SHARED((ACC_ROWS, 16), _F32),
            pltpu.SemaphoreType.DMA,
            pltpu.SemaphoreType.DMA,
            pltpu.SemaphoreType.DMA,
            pltpu.SemaphoreType.DMA,
            pltpu.SemaphoreType.DMA,
            pltpu.SemaphoreType.DMA,
        ],
    )
    return f(a2, b, src, dst, bnd)


def _a2_body(x_ref, bcol_ref, bfull_ref, u_ref, we1_ref, we3_ref, be_ref,
             out_ref, bnd_ref):
    i = pl.program_id(0)
    oh = (lax.broadcasted_iota(_I32, (NODE_BLK, N_GRAPHS), 1)
          == bcol_ref[...]).astype(_F32)
    uwe3 = jnp.dot(u_ref[...], we3_ref[...], preferred_element_type=_F32)
    out_ref[...] = (jnp.dot(x_ref[...], we1_ref[...], preferred_element_type=_F32)
                    + jnp.dot(oh, uwe3, preferred_element_type=_F32)
                    + be_ref[...])

    @pl.when(i == 0)
    def _():
        bf = bfull_ref[...].astype(_F32)
        c1 = jnp.sum((bf < 1.0).astype(_F32))
        c2 = jnp.sum((bf < 2.0).astype(_F32))
        c3 = jnp.sum((bf < 3.0).astype(_F32))
        row = lax.broadcasted_iota(_I32, (8, 128), 0)
        c1i = c1.astype(_I32)
        c2i = c2.astype(_I32)
        c3i = c3.astype(_I32)
        bnd_ref[...] = jnp.where(
            row == 0, c1i, jnp.where(row == 1, c2i,
                                     jnp.where(row == 2, c3i, 0)))


def _b_body(ea_ref, we2_ref, out_ref):
    out_ref[...] = jnp.dot(ea_ref[...], we2_ref[...], preferred_element_type=_F32)


def _node_body(x_ref, m0_ref, m1_ref, c0_ref, c1_ref, bcol_ref,
               u_ref, wn1_ref, wn2_ref, wn3_ref, bn_ref,
               xout_ref, xg_ref, nc_ref):
    i = pl.program_id(0)
    cnt = c0_ref[:, 0:1] + c1_ref[:, 0:1]
    mean = (m0_ref[...] + m1_ref[...]) / jnp.maximum(cnt, 1.0)
    oh = (lax.broadcasted_iota(_I32, (NODE_BLK, N_GRAPHS), 1)
          == bcol_ref[...]).astype(_F32)
    uwn3 = jnp.dot(u_ref[...], wn3_ref[...], preferred_element_type=_F32)
    acc = (jnp.dot(x_ref[...], wn1_ref[...], preferred_element_type=_F32)
           + jnp.dot(mean, wn2_ref[...], preferred_element_type=_F32)
           + jnp.dot(oh, uwn3, preferred_element_type=_F32)
           + bn_ref[...])
    xo = jnp.maximum(acc, 0.0)
    xout_ref[...] = xo

    @pl.when(i == 0)
    def _():
        xg_ref[...] = jnp.zeros_like(xg_ref)
        nc_ref[...] = jnp.zeros_like(nc_ref)

    dn = (((0,), (0,)), ((), ()))
    xg_ref[...] += lax.dot_general(oh, xo, dn, preferred_element_type=_F32)
    nc_ref[...] += lax.dot_general(oh, jnp.ones_like(xo), dn,
                                   preferred_element_type=_F32)


def _glob_body(egp_ref, ecp_ref, xg_ref, nc_ref,
               u_ref, wg1_ref, wg2_ref, wg3_ref, bg_ref, uout_ref):
    eg = jnp.sum(egp_ref[...], axis=0)
    ec = jnp.sum(jnp.sum(ecp_ref[...], axis=0), axis=1, keepdims=True)
    egm = eg / jnp.maximum(ec, 1.0)
    xgm = xg_ref[...] / jnp.maximum(nc_ref[...], 1.0)
    uo = (jnp.dot(egm, wg1_ref[...], preferred_element_type=_F32)
          + jnp.dot(xgm, wg2_ref[...], preferred_element_type=_F32)
          + jnp.dot(u_ref[...], wg3_ref[...], preferred_element_type=_F32)
          + bg_ref[...])
    uout_ref[...] = jnp.maximum(uo, 0.0)


def kernel(x, edge_attr, edge_index, u, batch, W_e, b_e, W_n, b_n, W_g, b_g):
    src = edge_index[0].astype(_I32)
    dst = edge_index[1].astype(_I32)
    bcol = batch.astype(_I32).reshape(N_NODES, 1)

    we1 = W_e[:D_NODE]
    we2 = W_e[D_NODE:D_NODE + D_EDGE]
    we3 = W_e[D_NODE + D_EDGE:]
    wn1 = W_n[:D_NODE]
    wn2 = W_n[D_NODE:2 * D_NODE]
    wn3 = W_n[2 * D_NODE:]
    wg1 = W_g[:D_NODE]
    wg2 = W_g[D_NODE:2 * D_NODE]
    wg3 = W_g[2 * D_NODE:]
    be = b_e.reshape(1, D_NODE)
    bn = b_n.reshape(1, D_NODE)
    bg = b_g.reshape(1, D_NODE)

    n_grid = N_NODES // NODE_BLK
    e_grid = N_EDGES // 4000

    full = lambda s: pl.BlockSpec(s, lambda i: tuple(0 for _ in s))

    a2, bnd = pl.pallas_call(
        _a2_body,
        grid=(n_grid,),
        in_specs=[
            pl.BlockSpec((NODE_BLK, D_NODE), lambda i: (i, 0)),
            pl.BlockSpec((NODE_BLK, 1), lambda i: (i, 0)),
            full((N_NODES, 1)),
            full((N_GRAPHS, D_GLOB)),
            full((D_NODE, D_NODE)),
            full((D_GLOB, D_NODE)),
            full((1, D_NODE)),
        ],
        out_specs=[
            pl.BlockSpec((NODE_BLK, D_NODE), lambda i: (i, 0)),
            pl.BlockSpec((8, 128), lambda i: (0, 0)),
        ],
        out_shape=[
            jax.ShapeDtypeStruct((N_NODES, D_NODE), _F32),
            jax.ShapeDtypeStruct((8, 128), _I32),
        ],
    )(x, bcol, bcol, u, we1, we3, be)

    b_edges = pl.pallas_call(
        _b_body,
        grid=(e_grid,),
        in_specs=[
            pl.BlockSpec((4000, D_EDGE), lambda i: (i, 0)),
            full((D_EDGE, D_NODE)),
        ],
        out_specs=pl.BlockSpec((4000, D_NODE), lambda i: (i, 0)),
        out_shape=jax.ShapeDtypeStruct((N_EDGES, D_NODE), _F32),
    )(edge_attr, we2)

    e, msg_pr, cnt_pr = _sc_edge(a2, b_edges, src, dst, bnd)
    msg_f = msg_pr.reshape(2, ACC_ROWS, D_NODE)
    cnt_f = cnt_pr.reshape(2, ACC_ROWS, 16)
    msg_p = msg_f[:, :N_NODES]
    cnt_p = cnt_f[:, :N_NODES]
    egp = msg_f[:, N_NODES:N_NODES + 64].reshape(N_WORKERS, N_GRAPHS, D_NODE)
    ecp = cnt_f[:, N_NODES:N_NODES + 64].reshape(N_WORKERS, N_GRAPHS, 16)

    x_out, xg, nc = pl.pallas_call(
        _node_body,
        grid=(n_grid,),
        in_specs=[
            pl.BlockSpec((NODE_BLK, D_NODE), lambda i: (i, 0)),
            pl.BlockSpec((NODE_BLK, D_NODE), lambda i: (i, 0)),
            pl.BlockSpec((NODE_BLK, D_NODE), lambda i: (i, 0)),
            pl.BlockSpec((NODE_BLK, 16), lambda i: (i, 0)),
            pl.BlockSpec((NODE_BLK, 16), lambda i: (i, 0)),
            pl.BlockSpec((NODE_BLK, 1), lambda i: (i, 0)),
            full((N_GRAPHS, D_GLOB)),
            full((D_NODE, D_NODE)),
            full((D_NODE, D_NODE)),
            full((D_GLOB, D_NODE)),
            full((1, D_NODE)),
        ],
        out_specs=[
            pl.BlockSpec((NODE_BLK, D_NODE), lambda i: (i, 0)),
            pl.BlockSpec((N_GRAPHS, D_NODE), lambda i: (0, 0)),
            pl.BlockSpec((N_GRAPHS, D_NODE), lambda i: (0, 0)),
        ],
        out_shape=[
            jax.ShapeDtypeStruct((N_NODES, D_NODE), _F32),
            jax.ShapeDtypeStruct((N_GRAPHS, D_NODE), _F32),
            jax.ShapeDtypeStruct((N_GRAPHS, D_NODE), _F32),
        ],
    )(x, msg_p[0], msg_p[1], cnt_p[0], cnt_p[1], bcol, u, wn1, wn2, wn3, bn)

    u_out = pl.pallas_call(
        _glob_body,
        grid=(1,),
        in_specs=[
            full((N_WORKERS, N_GRAPHS, D_NODE)),
            full((N_WORKERS, N_GRAPHS, 16)),
            full((N_GRAPHS, D_NODE)),
            full((N_GRAPHS, D_NODE)),
            full((N_GRAPHS, D_GLOB)),
            full((D_NODE, D_NODE)),
            full((D_NODE, D_NODE)),
            full((D_GLOB, D_NODE)),
            full((1, D_NODE)),
        ],
        out_specs=pl.BlockSpec((N_GRAPHS, D_NODE), lambda i: (0, 0)),
        out_shape=jax.ShapeDtypeStruct((N_GRAPHS, D_NODE), _F32),
    )(egp, ecp, xg, nc, u, wg1, wg2, wg3, bg)

    return (x_out, e, edge_index, u_out, batch)


# speedup vs baseline: 1.5962x; 1.1127x over previous
import jax
import jax.numpy as jnp
from jax import lax
from jax.experimental import pallas as pl
from jax.experimental.pallas import tpu as pltpu
from jax.experimental.pallas import tpu_sc as plsc

N_NODES = 10000
N_EDGES = 320000
D_NODE = 128
D_EDGE = 16
D_GLOB = 32
N_GRAPHS = 4

N_WORKERS = 32
EDGES_PER_TILE = N_EDGES // N_WORKERS
CHUNK = 80
N_CHUNKS = EDGES_PER_TILE // CHUNK
ACC_ROWS = 10240
TILE_ROWS = ACC_ROWS // 16

NODE_BLK = 1000

_F32 = jnp.float32
_I32 = jnp.int32


def _sc_edge_body(a2_hbm, b_hbm, src_hbm, dst_hbm, bnd_hbm,
                  e_hbm, msg_hbm, cnt_hbm,
                  src_v, dst_v, gidx, e_buf, b_v, ones_v, bnd_v,
                  msg_sh, cnt_sh, sem_g, sem_e, sem_m, sem_c, sem_x, sem_y,
                  sem_s, sem_d, sem_b):
    cid = lax.axis_index("c")
    sid = lax.axis_index("s")
    wid = cid * 16 + sid

    zero16 = jnp.zeros((16,), _F32)
    iota16 = lax.iota(_I32, 16)

    def _z_ones(i, c):
        ones_v[i, :] = zero16
        return c
    lax.fori_loop(0, CHUNK, _z_ones, 0)

    def _z_ebuf(i, c):
        for j in range(8):
            e_buf[i, pl.ds(j * 16, 16)] = zero16
        return c
    lax.fori_loop(0, CHUNK, _z_ebuf, 0)

    pltpu.sync_copy(bnd_hbm, bnd_v)

    rbase = sid * TILE_ROWS
    obase = cid * ACC_ROWS + rbase
    for k in range(8):
        pltpu.sync_copy(e_buf.at[pl.ds(0, 80)],
                        msg_sh.at[pl.ds(rbase + k * 80, 80)])
        pltpu.sync_copy(ones_v.at[pl.ds(0, 80)],
                        cnt_sh.at[pl.ds(rbase + k * 80, 80)])

    lane1 = jnp.where(iota16 == 0, jnp.float32(1.0), jnp.float32(0.0))

    def _s_ones(i, c):
        ones_v[i, :] = lane1
        return c
    lax.fori_loop(0, CHUNK, _s_ones, 0)

    plsc.subcore_barrier()

    ebase = wid * EDGES_PER_TILE
    one16 = jnp.full((16,), 1.0, _F32)

    def _chunk(t, c):
        base = ebase + t * CHUNK
        cps = pltpu.async_copy(src_hbm.at[pl.ds(base, CHUNK)], src_v, sem_s)
        cpd = pltpu.async_copy(dst_hbm.at[pl.ds(base, CHUNK)], dst_v.at[0],
                               sem_d)
        cpb = pltpu.async_copy(b_hbm.at[pl.ds(base, CHUNK)], b_v, sem_b)
        cps.wait()
        gcp = pltpu.async_copy(a2_hbm.at[src_v], e_buf, sem_g)
        cpd.wait()

        c1v = bnd_v[0, pl.ds(0, 16)]
        c2v = bnd_v[1, pl.ds(0, 16)]
        c3v = bnd_v[2, pl.ds(0, 16)]
        i1 = jnp.full((16,), 1, _I32)
        i0 = jnp.full((16,), 0, _I32)
        gb = N_NODES + sid * 4
        for grp in range(CHUNK // 16):
            s16 = src_v[pl.ds(grp * 16, 16)]
            g16 = (jnp.where(s16 >= c1v, i1, i0)
                   + jnp.where(s16 >= c2v, i1, i0)
                   + jnp.where(s16 >= c3v, i1, i0))
            gidx[0, pl.ds(grp * 16, 16)] = g16 + gb

        cpb.wait()
        gcp.wait()

        def _rows(i, cc):
            for j in range(8):
                v = e_buf[i, pl.ds(j * 16, 16)] + b_v[i, pl.ds(j * 16, 16)]
                v = jnp.maximum(v, 0.0)
                e_buf[i, pl.ds(j * 16, 16)] = v
            return cc
        lax.fori_loop(0, CHUNK, _rows, 0)

        cp_e = pltpu.async_copy(e_buf, e_hbm.at[pl.ds(base, CHUNK)], sem_e)
        cp_m = pltpu.async_copy(e_buf, msg_sh.at[dst_v.at[0]], sem_m, add=True)
        cp_c = pltpu.async_copy(ones_v, cnt_sh.at[dst_v.at[0]], sem_c, add=True)
        cp_x = pltpu.async_copy(e_buf, msg_sh.at[gidx.at[0]], sem_x, add=True)
        cp_y = pltpu.async_copy(ones_v, cnt_sh.at[gidx.at[0]], sem_y, add=True)
        cp_e.wait()
        cp_m.wait()
        cp_c.wait()
        cp_x.wait()
        cp_y.wait()
        return c

    lax.fori_loop(0, N_CHUNKS, _chunk, 0)

    plsc.subcore_barrier()

    for k in range(8):
        pltpu.sync_copy(msg_sh.at[pl.ds(rbase + k * 80, 80)], e_buf)
        pltpu.sync_copy(e_buf, msg_hbm.at[pl.ds(obase + k * 80, 80)])
        pltpu.sync_copy(cnt_sh.at[pl.ds(rbase + k * 80, 80)], ones_v)
        pltpu.sync_copy(ones_v, cnt_hbm.at[pl.ds(obase + k * 80, 80)])


def _sc_edge(a2, b, src, dst, bnd):
    f = pl.kernel(
        _sc_edge_body,
        out_type=[
            jax.ShapeDtypeStruct((N_EDGES, D_NODE), _F32),
            jax.ShapeDtypeStruct((2 * ACC_ROWS, D_NODE), _F32),
            jax.ShapeDtypeStruct((2 * ACC_ROWS, 16), _F32),
        ],
        mesh=plsc.VectorSubcoreMesh(core_axis_name="c", subcore_axis_name="s"),
        compiler_params=pltpu.CompilerParams(use_tc_tiling_on_sc=False,
                                            needs_layout_passes=False),
        scratch_types=[
            pltpu.VMEM((CHUNK,), _I32),
            pltpu.VMEM((1, CHUNK), _I32),
            pltpu.VMEM((1, CHUNK), _I32),
            pltpu.VMEM((CHUNK, D_NODE), _F32),
            pltpu.VMEM((CHUNK, D_NODE), _F32),
            pltpu.VMEM((CHUNK, 16), _F32),
            pltpu.VMEM((8, 128), _I32),
            pltpu.VMEM_SHARED((ACC_ROWS, D_NODE), _F32),
            pltpu.VMEM_SHARED((ACC_ROWS, 16), _F32),
            pltpu.SemaphoreType.DMA,
            pltpu.SemaphoreType.DMA,
            pltpu.SemaphoreType.DMA,
            pltpu.SemaphoreType.DMA,
            pltpu.SemaphoreType.DMA,
            pltpu.SemaphoreType.DMA,
            pltpu.SemaphoreType.DMA,
            pltpu.SemaphoreType.DMA,
            pltpu.SemaphoreType.DMA,
        ],
    )
    return f(a2, b, src, dst, bnd)


def _a2_body(x_ref, bcol_ref, bfull_ref, u_ref, we1_ref, we3_ref, be_ref,
             out_ref, bnd_ref):
    i = pl.program_id(0)
    oh = (lax.broadcasted_iota(_I32, (NODE_BLK, N_GRAPHS), 1)
          == bcol_ref[...]).astype(_F32)
    uwe3 = jnp.dot(u_ref[...], we3_ref[...], preferred_element_type=_F32)
    out_ref[...] = (jnp.dot(x_ref[...], we1_ref[...], preferred_element_type=_F32)
                    + jnp.dot(oh, uwe3, preferred_element_type=_F32)
                    + be_ref[...])

    @pl.when(i == 0)
    def _():
        bf = bfull_ref[...].astype(_F32)
        c1 = jnp.sum((bf < 1.0).astype(_F32))
        c2 = jnp.sum((bf < 2.0).astype(_F32))
        c3 = jnp.sum((bf < 3.0).astype(_F32))
        row = lax.broadcasted_iota(_I32, (8, 128), 0)
        c1i = c1.astype(_I32)
        c2i = c2.astype(_I32)
        c3i = c3.astype(_I32)
        bnd_ref[...] = jnp.where(
            row == 0, c1i, jnp.where(row == 1, c2i,
                                     jnp.where(row == 2, c3i, 0)))


def _b_body(ea_ref, we2_ref, out_ref):
    out_ref[...] = jnp.dot(ea_ref[...], we2_ref[...], preferred_element_type=_F32)


def _node_body(x_ref, m0_ref, m1_ref, c0_ref, c1_ref, bcol_ref,
               u_ref, wn1_ref, wn2_ref, wn3_ref, bn_ref,
               xout_ref, xg_ref, nc_ref):
    i = pl.program_id(0)
    cnt = c0_ref[:, 0:1] + c1_ref[:, 0:1]
    mean = (m0_ref[...] + m1_ref[...]) / jnp.maximum(cnt, 1.0)
    oh = (lax.broadcasted_iota(_I32, (NODE_BLK, N_GRAPHS), 1)
          == bcol_ref[...]).astype(_F32)
    uwn3 = jnp.dot(u_ref[...], wn3_ref[...], preferred_element_type=_F32)
    acc = (jnp.dot(x_ref[...], wn1_ref[...], preferred_element_type=_F32)
           + jnp.dot(mean, wn2_ref[...], preferred_element_type=_F32)
           + jnp.dot(oh, uwn3, preferred_element_type=_F32)
           + bn_ref[...])
    xo = jnp.maximum(acc, 0.0)
    xout_ref[...] = xo

    @pl.when(i == 0)
    def _():
        xg_ref[...] = jnp.zeros_like(xg_ref)
        nc_ref[...] = jnp.zeros_like(nc_ref)

    dn = (((0,), (0,)), ((), ()))
    xg_ref[...] += lax.dot_general(oh, xo, dn, preferred_element_type=_F32)
    nc_ref[...] += lax.dot_general(oh, jnp.ones_like(xo), dn,
                                   preferred_element_type=_F32)


def _glob_body(egp_ref, ecp_ref, xg_ref, nc_ref,
               u_ref, wg1_ref, wg2_ref, wg3_ref, bg_ref, uout_ref):
    eg = jnp.sum(egp_ref[...], axis=0)
    ec = jnp.sum(jnp.sum(ecp_ref[...], axis=0), axis=1, keepdims=True)
    egm = eg / jnp.maximum(ec, 1.0)
    xgm = xg_ref[...] / jnp.maximum(nc_ref[...], 1.0)
    uo = (jnp.dot(egm, wg1_ref[...], preferred_element_type=_F32)
          + jnp.dot(xgm, wg2_ref[...], preferred_element_type=_F32)
          + jnp.dot(u_ref[...], wg3_ref[...], preferred_element_type=_F32)
          + bg_ref[...])
    uout_ref[...] = jnp.maximum(uo, 0.0)


def kernel(x, edge_attr, edge_index, u, batch, W_e, b_e, W_n, b_n, W_g, b_g):
    src = edge_index[0].astype(_I32)
    dst = edge_index[1].astype(_I32)
    bcol = batch.astype(_I32).reshape(N_NODES, 1)

    we1 = W_e[:D_NODE]
    we2 = W_e[D_NODE:D_NODE + D_EDGE]
    we3 = W_e[D_NODE + D_EDGE:]
    wn1 = W_n[:D_NODE]
    wn2 = W_n[D_NODE:2 * D_NODE]
    wn3 = W_n[2 * D_NODE:]
    wg1 = W_g[:D_NODE]
    wg2 = W_g[D_NODE:2 * D_NODE]
    wg3 = W_g[2 * D_NODE:]
    be = b_e.reshape(1, D_NODE)
    bn = b_n.reshape(1, D_NODE)
    bg = b_g.reshape(1, D_NODE)

    n_grid = N_NODES // NODE_BLK
    e_grid = N_EDGES // 4000

    full = lambda s: pl.BlockSpec(s, lambda i: tuple(0 for _ in s))

    a2, bnd = pl.pallas_call(
        _a2_body,
        grid=(n_grid,),
        in_specs=[
            pl.BlockSpec((NODE_BLK, D_NODE), lambda i: (i, 0)),
            pl.BlockSpec((NODE_BLK, 1), lambda i: (i, 0)),
            full((N_NODES, 1)),
            full((N_GRAPHS, D_GLOB)),
            full((D_NODE, D_NODE)),
            full((D_GLOB, D_NODE)),
            full((1, D_NODE)),
        ],
        out_specs=[
            pl.BlockSpec((NODE_BLK, D_NODE), lambda i: (i, 0)),
            pl.BlockSpec((8, 128), lambda i: (0, 0)),
        ],
        out_shape=[
            jax.ShapeDtypeStruct((N_NODES, D_NODE), _F32),
            jax.ShapeDtypeStruct((8, 128), _I32),
        ],
    )(x, bcol, bcol, u, we1, we3, be)

    b_edges = pl.pallas_call(
        _b_body,
        grid=(e_grid,),
        in_specs=[
            pl.BlockSpec((4000, D_EDGE), lambda i: (i, 0)),
            full((D_EDGE, D_NODE)),
        ],
        out_specs=pl.BlockSpec((4000, D_NODE), lambda i: (i, 0)),
        out_shape=jax.ShapeDtypeStruct((N_EDGES, D_NODE), _F32),
    )(edge_attr, we2)

    e, msg_pr, cnt_pr = _sc_edge(a2, b_edges, src, dst, bnd)
    msg_f = msg_pr.reshape(2, ACC_ROWS, D_NODE)
    cnt_f = cnt_pr.reshape(2, ACC_ROWS, 16)
    msg_p = msg_f[:, :N_NODES]
    cnt_p = cnt_f[:, :N_NODES]
    egp = msg_f[:, N_NODES:N_NODES + 64].reshape(N_WORKERS, N_GRAPHS, D_NODE)
    ecp = cnt_f[:, N_NODES:N_NODES + 64].reshape(N_WORKERS, N_GRAPHS, 16)

    x_out, xg, nc = pl.pallas_call(
        _node_body,
        grid=(n_grid,),
        in_specs=[
            pl.BlockSpec((NODE_BLK, D_NODE), lambda i: (i, 0)),
            pl.BlockSpec((NODE_BLK, D_NODE), lambda i: (i, 0)),
            pl.BlockSpec((NODE_BLK, D_NODE), lambda i: (i, 0)),
            pl.BlockSpec((NODE_BLK, 16), lambda i: (i, 0)),
            pl.BlockSpec((NODE_BLK, 16), lambda i: (i, 0)),
            pl.BlockSpec((NODE_BLK, 1), lambda i: (i, 0)),
            full((N_GRAPHS, D_GLOB)),
            full((D_NODE, D_NODE)),
            full((D_NODE, D_NODE)),
            full((D_GLOB, D_NODE)),
            full((1, D_NODE)),
        ],
        out_specs=[
            pl.BlockSpec((NODE_BLK, D_NODE), lambda i: (i, 0)),
            pl.BlockSpec((N_GRAPHS, D_NODE), lambda i: (0, 0)),
            pl.BlockSpec((N_GRAPHS, D_NODE), lambda i: (0, 0)),
        ],
        out_shape=[
            jax.ShapeDtypeStruct((N_NODES, D_NODE), _F32),
            jax.ShapeDtypeStruct((N_GRAPHS, D_NODE), _F32),
            jax.ShapeDtypeStruct((N_GRAPHS, D_NODE), _F32),
        ],
    )(x, msg_p[0], msg_p[1], cnt_p[0], cnt_p[1], bcol, u, wn1, wn2, wn3, bn)

    u_out = pl.pallas_call(
        _glob_body,
        grid=(1,),
        in_specs=[
            full((N_WORKERS, N_GRAPHS, D_NODE)),
            full((N_WORKERS, N_GRAPHS, 16)),
            full((N_GRAPHS, D_NODE)),
            full((N_GRAPHS, D_NODE)),
            full((N_GRAPHS, D_GLOB)),
            full((D_NODE, D_NODE)),
            full((D_NODE, D_NODE)),
            full((D_GLOB, D_NODE)),
            full((1, D_NODE)),
        ],
        out_specs=pl.BlockSpec((N_GRAPHS, D_NODE), lambda i: (0, 0)),
        out_shape=jax.ShapeDtypeStruct((N_GRAPHS, D_NODE), _F32),
    )(egp, ecp, xg, nc, u, wg1, wg2, wg3, bg)

    return (x_out, e, edge_index, u_out, batch)


# speedup vs baseline: 1.6484x; 1.0327x over previous
import jax
import jax.numpy as jnp
from jax import lax
from jax.experimental import pallas as pl
from jax.experimental.pallas import tpu as pltpu
from jax.experimental.pallas import tpu_sc as plsc

N_NODES = 10000
N_EDGES = 320000
D_NODE = 128
D_EDGE = 16
D_GLOB = 32
N_GRAPHS = 4

N_WORKERS = 32
EDGES_PER_TILE = N_EDGES // N_WORKERS
CHUNK = 80
N_CHUNKS = EDGES_PER_TILE // CHUNK
ACC_ROWS = 10240
TILE_ROWS = ACC_ROWS // 16

NODE_BLK = 1000

_F32 = jnp.float32
_I32 = jnp.int32


def _sc_edge_body(a2_hbm, b_hbm, src_hbm, dst_hbm, bnd_hbm,
                  e_hbm, msg_hbm, cnt_hbm,
                  src_v, dst_v, src_v2, dst_v2, gidx, e_buf, b_v, ones_v,
                  bnd_v,
                  msg_sh, cnt_sh, sem_g, sem_e, sem_m, sem_c, sem_x, sem_y,
                  sem_s, sem_d, sem_b):
    cid = lax.axis_index("c")
    sid = lax.axis_index("s")
    wid = cid * 16 + sid

    zero16 = jnp.zeros((16,), _F32)
    iota16 = lax.iota(_I32, 16)

    def _z_ones(i, c):
        ones_v[i, :] = zero16
        return c
    lax.fori_loop(0, CHUNK, _z_ones, 0)

    def _z_ebuf(i, c):
        for j in range(8):
            e_buf[i, pl.ds(j * 16, 16)] = zero16
        return c
    lax.fori_loop(0, CHUNK, _z_ebuf, 0)

    pltpu.sync_copy(bnd_hbm, bnd_v)

    rbase = sid * TILE_ROWS
    obase = cid * ACC_ROWS + rbase
    for k in range(8):
        pltpu.sync_copy(e_buf.at[pl.ds(0, 80)],
                        msg_sh.at[pl.ds(rbase + k * 80, 80)])
        pltpu.sync_copy(ones_v.at[pl.ds(0, 80)],
                        cnt_sh.at[pl.ds(rbase + k * 80, 80)])

    lane1 = jnp.where(iota16 == 0, jnp.float32(1.0), jnp.float32(0.0))

    def _s_ones(i, c):
        ones_v[i, :] = lane1
        return c
    lax.fori_loop(0, CHUNK, _s_ones, 0)

    plsc.subcore_barrier()

    ebase = wid * EDGES_PER_TILE
    one16 = jnp.full((16,), 1.0, _F32)

    def _fire_idx(t, sv, dv):
        base = ebase + t * CHUNK
        pltpu.async_copy(src_hbm.at[pl.ds(base, CHUNK)], sv, sem_s)
        pltpu.async_copy(dst_hbm.at[pl.ds(base, CHUNK)], dv.at[0], sem_d)

    def _wait_idx(t, sv, dv):
        base = ebase + t * CHUNK
        pltpu.make_async_copy(src_hbm.at[pl.ds(base, CHUNK)], sv, sem_s).wait()
        pltpu.make_async_copy(dst_hbm.at[pl.ds(base, CHUNK)], dv.at[0],
                              sem_d).wait()

    def _process(t, sv, dv, pt, psv, pdv, prefetch):
        base = ebase + t * CHUNK
        cpb = pltpu.async_copy(b_hbm.at[pl.ds(base, CHUNK)], b_v, sem_b)
        _wait_idx(t, sv, dv)
        gcp = pltpu.async_copy(a2_hbm.at[sv], e_buf, sem_g)
        if prefetch:
            _fire_idx(pt, psv, pdv)

        c1v = bnd_v[0, pl.ds(0, 16)]
        c2v = bnd_v[1, pl.ds(0, 16)]
        c3v = bnd_v[2, pl.ds(0, 16)]
        i1 = jnp.full((16,), 1, _I32)
        i0 = jnp.full((16,), 0, _I32)
        gb = N_NODES + sid * 4
        for grp in range(CHUNK // 16):
            s16 = sv[pl.ds(grp * 16, 16)]
            g16 = (jnp.where(s16 >= c1v, i1, i0)
                   + jnp.where(s16 >= c2v, i1, i0)
                   + jnp.where(s16 >= c3v, i1, i0))
            gidx[0, pl.ds(grp * 16, 16)] = g16 + gb

        cpb.wait()
        gcp.wait()

        def _rows(i, cc):
            for j in range(8):
                v = e_buf[i, pl.ds(j * 16, 16)] + b_v[i, pl.ds(j * 16, 16)]
                v = jnp.maximum(v, 0.0)
                e_buf[i, pl.ds(j * 16, 16)] = v
            return cc
        lax.fori_loop(0, CHUNK, _rows, 0)

        cp_e = pltpu.async_copy(e_buf, e_hbm.at[pl.ds(base, CHUNK)], sem_e)
        cp_m = pltpu.async_copy(e_buf, msg_sh.at[dv.at[0]], sem_m, add=True)
        cp_c = pltpu.async_copy(ones_v, cnt_sh.at[dv.at[0]], sem_c, add=True)
        cp_x = pltpu.async_copy(e_buf, msg_sh.at[gidx.at[0]], sem_x, add=True)
        cp_y = pltpu.async_copy(ones_v, cnt_sh.at[gidx.at[0]], sem_y, add=True)
        cp_e.wait()
        cp_m.wait()
        cp_c.wait()
        cp_x.wait()
        cp_y.wait()

    _fire_idx(0, src_v, dst_v)

    def _pair(k, c):
        t0 = 2 * k
        _process(t0, src_v, dst_v, t0 + 1, src_v2, dst_v2, True)
        _process(t0 + 1, src_v2, dst_v2, t0 + 2, src_v, dst_v, True)
        return c

    lax.fori_loop(0, (N_CHUNKS - 1) // 2, _pair, 0)
    _process(N_CHUNKS - 1, src_v, dst_v, 0, src_v2, dst_v2, False)

    plsc.subcore_barrier()

    for k in range(8):
        pltpu.sync_copy(msg_sh.at[pl.ds(rbase + k * 80, 80)], e_buf)
        pltpu.sync_copy(e_buf, msg_hbm.at[pl.ds(obase + k * 80, 80)])
        pltpu.sync_copy(cnt_sh.at[pl.ds(rbase + k * 80, 80)], ones_v)
        pltpu.sync_copy(ones_v, cnt_hbm.at[pl.ds(obase + k * 80, 80)])


def _sc_edge(a2, b, src, dst, bnd):
    f = pl.kernel(
        _sc_edge_body,
        out_type=[
            jax.ShapeDtypeStruct((N_EDGES, D_NODE), _F32),
            jax.ShapeDtypeStruct((2 * ACC_ROWS, D_NODE), _F32),
            jax.ShapeDtypeStruct((2 * ACC_ROWS, 16), _F32),
        ],
        mesh=plsc.VectorSubcoreMesh(core_axis_name="c", subcore_axis_name="s"),
        compiler_params=pltpu.CompilerParams(use_tc_tiling_on_sc=False,
                                            needs_layout_passes=False),
        scratch_types=[
            pltpu.VMEM((CHUNK,), _I32),
            pltpu.VMEM((1, CHUNK), _I32),
            pltpu.VMEM((CHUNK,), _I32),
            pltpu.VMEM((1, CHUNK), _I32),
            pltpu.VMEM((1, CHUNK), _I32),
            pltpu.VMEM((CHUNK, D_NODE), _F32),
            pltpu.VMEM((CHUNK, D_NODE), _F32),
            pltpu.VMEM((CHUNK, 16), _F32),
            pltpu.VMEM((8, 128), _I32),
            pltpu.VMEM_SHARED((ACC_ROWS, D_NODE), _F32),
            pltpu.VMEM_SHARED((ACC_ROWS, 16), _F32),
            pltpu.SemaphoreType.DMA,
            pltpu.SemaphoreType.DMA,
            pltpu.SemaphoreType.DMA,
            pltpu.SemaphoreType.DMA,
            pltpu.SemaphoreType.DMA,
            pltpu.SemaphoreType.DMA,
            pltpu.SemaphoreType.DMA,
            pltpu.SemaphoreType.DMA,
            pltpu.SemaphoreType.DMA,
        ],
    )
    return f(a2, b, src, dst, bnd)


def _a2_body(x_ref, bcol_ref, bfull_ref, u_ref, we1_ref, we3_ref, be_ref,
             out_ref, bnd_ref):
    i = pl.program_id(0)
    oh = (lax.broadcasted_iota(_I32, (NODE_BLK, N_GRAPHS), 1)
          == bcol_ref[...]).astype(_F32)
    uwe3 = jnp.dot(u_ref[...], we3_ref[...], preferred_element_type=_F32)
    out_ref[...] = (jnp.dot(x_ref[...], we1_ref[...], preferred_element_type=_F32)
                    + jnp.dot(oh, uwe3, preferred_element_type=_F32)
                    + be_ref[...])

    @pl.when(i == 0)
    def _():
        bf = bfull_ref[...].astype(_F32)
        c1 = jnp.sum((bf < 1.0).astype(_F32))
        c2 = jnp.sum((bf < 2.0).astype(_F32))
        c3 = jnp.sum((bf < 3.0).astype(_F32))
        row = lax.broadcasted_iota(_I32, (8, 128), 0)
        c1i = c1.astype(_I32)
        c2i = c2.astype(_I32)
        c3i = c3.astype(_I32)
        bnd_ref[...] = jnp.where(
            row == 0, c1i, jnp.where(row == 1, c2i,
                                     jnp.where(row == 2, c3i, 0)))


def _b_body(ea_ref, we2_ref, out_ref):
    out_ref[...] = jnp.dot(ea_ref[...], we2_ref[...], preferred_element_type=_F32)


def _node_body(x_ref, m0_ref, m1_ref, c0_ref, c1_ref, bcol_ref,
               u_ref, wn1_ref, wn2_ref, wn3_ref, bn_ref,
               xout_ref, xg_ref, nc_ref):
    i = pl.program_id(0)
    cnt = c0_ref[:, 0:1] + c1_ref[:, 0:1]
    mean = (m0_ref[...] + m1_ref[...]) / jnp.maximum(cnt, 1.0)
    oh = (lax.broadcasted_iota(_I32, (NODE_BLK, N_GRAPHS), 1)
          == bcol_ref[...]).astype(_F32)
    uwn3 = jnp.dot(u_ref[...], wn3_ref[...], preferred_element_type=_F32)
    acc = (jnp.dot(x_ref[...], wn1_ref[...], preferred_element_type=_F32)
           + jnp.dot(mean, wn2_ref[...], preferred_element_type=_F32)
           + jnp.dot(oh, uwn3, preferred_element_type=_F32)
           + bn_ref[...])
    xo = jnp.maximum(acc, 0.0)
    xout_ref[...] = xo

    @pl.when(i == 0)
    def _():
        xg_ref[...] = jnp.zeros_like(xg_ref)
        nc_ref[...] = jnp.zeros_like(nc_ref)

    dn = (((0,), (0,)), ((), ()))
    xg_ref[...] += lax.dot_general(oh, xo, dn, preferred_element_type=_F32)
    nc_ref[...] += lax.dot_general(oh, jnp.ones_like(xo), dn,
                                   preferred_element_type=_F32)


def _glob_body(egp_ref, ecp_ref, xg_ref, nc_ref,
               u_ref, wg1_ref, wg2_ref, wg3_ref, bg_ref, uout_ref):
    eg = jnp.sum(egp_ref[...], axis=0)
    ec = jnp.sum(jnp.sum(ecp_ref[...], axis=0), axis=1, keepdims=True)
    egm = eg / jnp.maximum(ec, 1.0)
    xgm = xg_ref[...] / jnp.maximum(nc_ref[...], 1.0)
    uo = (jnp.dot(egm, wg1_ref[...], preferred_element_type=_F32)
          + jnp.dot(xgm, wg2_ref[...], preferred_element_type=_F32)
          + jnp.dot(u_ref[...], wg3_ref[...], preferred_element_type=_F32)
          + bg_ref[...])
    uout_ref[...] = jnp.maximum(uo, 0.0)


def kernel(x, edge_attr, edge_index, u, batch, W_e, b_e, W_n, b_n, W_g, b_g):
    src = edge_index[0].astype(_I32)
    dst = edge_index[1].astype(_I32)
    bcol = batch.astype(_I32).reshape(N_NODES, 1)

    we1 = W_e[:D_NODE]
    we2 = W_e[D_NODE:D_NODE + D_EDGE]
    we3 = W_e[D_NODE + D_EDGE:]
    wn1 = W_n[:D_NODE]
    wn2 = W_n[D_NODE:2 * D_NODE]
    wn3 = W_n[2 * D_NODE:]
    wg1 = W_g[:D_NODE]
    wg2 = W_g[D_NODE:2 * D_NODE]
    wg3 = W_g[2 * D_NODE:]
    be = b_e.reshape(1, D_NODE)
    bn = b_n.reshape(1, D_NODE)
    bg = b_g.reshape(1, D_NODE)

    n_grid = N_NODES // NODE_BLK
    e_grid = N_EDGES // 4000

    full = lambda s: pl.BlockSpec(s, lambda i: tuple(0 for _ in s))

    a2, bnd = pl.pallas_call(
        _a2_body,
        grid=(n_grid,),
        in_specs=[
            pl.BlockSpec((NODE_BLK, D_NODE), lambda i: (i, 0)),
            pl.BlockSpec((NODE_BLK, 1), lambda i: (i, 0)),
            full((N_NODES, 1)),
            full((N_GRAPHS, D_GLOB)),
            full((D_NODE, D_NODE)),
            full((D_GLOB, D_NODE)),
            full((1, D_NODE)),
        ],
        out_specs=[
            pl.BlockSpec((NODE_BLK, D_NODE), lambda i: (i, 0)),
            pl.BlockSpec((8, 128), lambda i: (0, 0)),
        ],
        out_shape=[
            jax.ShapeDtypeStruct((N_NODES, D_NODE), _F32),
            jax.ShapeDtypeStruct((8, 128), _I32),
        ],
    )(x, bcol, bcol, u, we1, we3, be)

    b_edges = pl.pallas_call(
        _b_body,
        grid=(e_grid,),
        in_specs=[
            pl.BlockSpec((4000, D_EDGE), lambda i: (i, 0)),
            full((D_EDGE, D_NODE)),
        ],
        out_specs=pl.BlockSpec((4000, D_NODE), lambda i: (i, 0)),
        out_shape=jax.ShapeDtypeStruct((N_EDGES, D_NODE), _F32),
    )(edge_attr, we2)

    e, msg_pr, cnt_pr = _sc_edge(a2, b_edges, src, dst, bnd)
    msg_f = msg_pr.reshape(2, ACC_ROWS, D_NODE)
    cnt_f = cnt_pr.reshape(2, ACC_ROWS, 16)
    msg_p = msg_f[:, :N_NODES]
    cnt_p = cnt_f[:, :N_NODES]
    egp = msg_f[:, N_NODES:N_NODES + 64].reshape(N_WORKERS, N_GRAPHS, D_NODE)
    ecp = cnt_f[:, N_NODES:N_NODES + 64].reshape(N_WORKERS, N_GRAPHS, 16)

    x_out, xg, nc = pl.pallas_call(
        _node_body,
        grid=(n_grid,),
        in_specs=[
            pl.BlockSpec((NODE_BLK, D_NODE), lambda i: (i, 0)),
            pl.BlockSpec((NODE_BLK, D_NODE), lambda i: (i, 0)),
            pl.BlockSpec((NODE_BLK, D_NODE), lambda i: (i, 0)),
            pl.BlockSpec((NODE_BLK, 16), lambda i: (i, 0)),
            pl.BlockSpec((NODE_BLK, 16), lambda i: (i, 0)),
            pl.BlockSpec((NODE_BLK, 1), lambda i: (i, 0)),
            full((N_GRAPHS, D_GLOB)),
            full((D_NODE, D_NODE)),
            full((D_NODE, D_NODE)),
            full((D_GLOB, D_NODE)),
            full((1, D_NODE)),
        ],
        out_specs=[
            pl.BlockSpec((NODE_BLK, D_NODE), lambda i: (i, 0)),
            pl.BlockSpec((N_GRAPHS, D_NODE), lambda i: (0, 0)),
            pl.BlockSpec((N_GRAPHS, D_NODE), lambda i: (0, 0)),
        ],
        out_shape=[
            jax.ShapeDtypeStruct((N_NODES, D_NODE), _F32),
            jax.ShapeDtypeStruct((N_GRAPHS, D_NODE), _F32),
            jax.ShapeDtypeStruct((N_GRAPHS, D_NODE), _F32),
        ],
    )(x, msg_p[0], msg_p[1], cnt_p[0], cnt_p[1], bcol, u, wn1, wn2, wn3, bn)

    u_out = pl.pallas_call(
        _glob_body,
        grid=(1,),
        in_specs=[
            full((N_WORKERS, N_GRAPHS, D_NODE)),
            full((N_WORKERS, N_GRAPHS, 16)),
            full((N_GRAPHS, D_NODE)),
            full((N_GRAPHS, D_NODE)),
            full((N_GRAPHS, D_GLOB)),
            full((D_NODE, D_NODE)),
            full((D_NODE, D_NODE)),
            full((D_GLOB, D_NODE)),
            full((1, D_NODE)),
        ],
        out_specs=pl.BlockSpec((N_GRAPHS, D_NODE), lambda i: (0, 0)),
        out_shape=jax.ShapeDtypeStruct((N_GRAPHS, D_NODE), _F32),
    )(egp, ecp, xg, nc, u, wg1, wg2, wg3, bg)

    return (x_out, e, edge_index, u_out, batch)
